# baseline (speedup 1.0000x reference)
import functools

import jax
import jax.numpy as jnp
from jax import lax
from jax.experimental import pallas as pl
from jax.experimental.pallas import tpu as pltpu
from jax.experimental.pallas import tpu_sc as plsc

VOCAB = 100
HIDDEN = 1024
BATCH = 4096
SEQ = 20
NUM_ROWS = BATCH * SEQ
NUM_CORES = 2
NUM_SUBCORES = 16
NUM_WORKERS = NUM_CORES * NUM_SUBCORES
ROWS_PER_WORKER = NUM_ROWS // NUM_WORKERS
CHUNK = 40
NBUF = 2
NUM_CHUNKS = ROWS_PER_WORKER // CHUNK
REPL = 8

_MESH = plsc.VectorSubcoreMesh(core_axis_name="c", subcore_axis_name="s")


@functools.partial(
    pl.kernel,
    out_type=jax.ShapeDtypeStruct((NUM_ROWS, HIDDEN), jnp.float32),
    mesh=_MESH,
    scratch_types=[
        pltpu.VMEM((ROWS_PER_WORKER,), jnp.int32),
        pltpu.VMEM((NBUF, CHUNK, HIDDEN), jnp.float32),
        [pltpu.SemaphoreType.DMA] * NBUF,
        [pltpu.SemaphoreType.DMA] * NBUF,
    ],
)
def _emb_gather(idx_hbm, table_hbm, out_hbm, idx_v, bufs, gsems, osems):
    wid = lax.axis_index("s") * NUM_CORES + lax.axis_index("c")
    base = wid * ROWS_PER_WORKER
    pltpu.sync_copy(idx_hbm.at[pl.ds(base, ROWS_PER_WORKER)], idx_v)

    def out_slice(g):
        return out_hbm.at[pl.ds(base + g * CHUNK, CHUNK)]

    def body(step, carry):
        g0 = step * NBUF
        for b in range(NBUF):
            g = g0 + b

            @pl.when(g >= NBUF)
            def _():
                pltpu.make_async_copy(
                    bufs.at[b], out_slice(g - NBUF), osems[b]).wait()

            pltpu.async_copy(
                table_hbm.at[idx_v.at[pl.ds(g * CHUNK, CHUNK)]],
                bufs.at[b], gsems[b]).wait()

            pltpu.async_copy(bufs.at[b], out_slice(g), osems[b])
        return carry

    lax.fori_loop(0, NUM_CHUNKS // NBUF, body, 0)

    for b in range(NBUF):
        g = NUM_CHUNKS - NBUF + b
        pltpu.make_async_copy(bufs.at[b], out_slice(g), osems[b]).wait()


def kernel(indices, word_embeddings):
    idx_t = indices.T.reshape(NUM_ROWS).astype(jnp.int32)
    repl = jnp.arange(NUM_ROWS, dtype=jnp.int32) % REPL
    idx_t = idx_t + repl * VOCAB
    table_rep = jnp.tile(word_embeddings, (REPL, 1))
    flat = _emb_gather(idx_t, table_rep)
    return flat.reshape(SEQ, BATCH, HIDDEN).transpose(1, 0, 2)


# speedup vs baseline: 1.0138x; 1.0138x over previous
import functools

import jax
import jax.numpy as jnp
from jax import lax
from jax.experimental import pallas as pl
from jax.experimental.pallas import tpu as pltpu
from jax.experimental.pallas import tpu_sc as plsc

VOCAB = 100
HIDDEN = 1024
BATCH = 4096
SEQ = 20
NUM_ROWS = BATCH * SEQ
NUM_CORES = 2
NUM_SUBCORES = 16
NUM_WORKERS = NUM_CORES * NUM_SUBCORES
ROWS_PER_WORKER = NUM_ROWS // NUM_WORKERS
CHUNK = 40
NBUF = 2
NUM_CHUNKS = ROWS_PER_WORKER // CHUNK
REPL = 16

_MESH = plsc.VectorSubcoreMesh(core_axis_name="c", subcore_axis_name="s")


@functools.partial(
    pl.kernel,
    out_type=jax.ShapeDtypeStruct((NUM_ROWS, HIDDEN), jnp.float32),
    mesh=_MESH,
    scratch_types=[
        pltpu.VMEM((ROWS_PER_WORKER,), jnp.int32),
        pltpu.VMEM((NBUF, CHUNK, HIDDEN), jnp.float32),
        [pltpu.SemaphoreType.DMA] * NBUF,
        [pltpu.SemaphoreType.DMA] * NBUF,
    ],
)
def _emb_gather(idx_hbm, table_hbm, out_hbm, idx_v, bufs, gsems, osems):
    wid = lax.axis_index("s") * NUM_CORES + lax.axis_index("c")
    base = wid * ROWS_PER_WORKER
    pltpu.sync_copy(idx_hbm.at[pl.ds(base, ROWS_PER_WORKER)], idx_v)

    def out_slice(g):
        return out_hbm.at[pl.ds(base + g * CHUNK, CHUNK)]

    def body(step, carry):
        g0 = step * NBUF
        for b in range(NBUF):
            g = g0 + b

            @pl.when(g >= NBUF)
            def _():
                pltpu.make_async_copy(
                    bufs.at[b], out_slice(g - NBUF), osems[b]).wait()

            pltpu.async_copy(
                table_hbm.at[idx_v.at[pl.ds(g * CHUNK, CHUNK)]],
                bufs.at[b], gsems[b]).wait()

            pltpu.async_copy(bufs.at[b], out_slice(g), osems[b])
        return carry

    lax.fori_loop(0, NUM_CHUNKS // NBUF, body, 0)

    for b in range(NBUF):
        g = NUM_CHUNKS - NBUF + b
        pltpu.make_async_copy(bufs.at[b], out_slice(g), osems[b]).wait()


def kernel(indices, word_embeddings):
    idx_t = indices.T.reshape(NUM_ROWS).astype(jnp.int32)
    repl = jnp.arange(NUM_ROWS, dtype=jnp.int32) % REPL
    idx_t = idx_t + repl * VOCAB
    table_rep = jnp.tile(word_embeddings, (REPL, 1))
    flat = _emb_gather(idx_t, table_rep)
    return flat.reshape(SEQ, BATCH, HIDDEN).transpose(1, 0, 2)
